# Initial kernel scaffold; baseline (speedup 1.0000x reference)
import jax
import jax.numpy as jnp
from jax.experimental import pallas as pl


def kernel(x, edge_index, W, b):
    raise NotImplementedError("write your pallas kernel here")


# speedup vs baseline: 3.8241x; 3.8241x over previous
import functools

import jax
import jax.numpy as jnp
from jax import lax
from jax.experimental import pallas as pl
from jax.experimental.pallas import tpu as pltpu
from jax.experimental.pallas import tpu_sc as plsc

D = 128
NUM_NODES = 10000

NC = 2
NS = 16
NW = NC * NS

CHUNK = 128
CHUNKS_PER_WORKER = 80
EDGES_PER_WORKER = CHUNK * CHUNKS_PER_WORKER
E_PAD = EDGES_PER_WORKER * NW
AGG_ROWS = 10240
ROWS_PER_TILE = AGG_ROWS // NS
DUMMY_ROW = NUM_NODES

ZBLK = 64


def _linear_body(x_ref, wt_ref, b_ref, h_ref):
    h_ref[...] = (
        jnp.dot(x_ref[...], wt_ref[...], preferred_element_type=jnp.float32)
        + b_ref[...]
    )


def _linear(x, wt, b2):
    m = x.shape[0]
    bm = 1000
    return pl.pallas_call(
        _linear_body,
        grid=(m // bm,),
        in_specs=[
            pl.BlockSpec((bm, D), lambda i: (i, 0)),
            pl.BlockSpec((D, D), lambda i: (0, 0)),
            pl.BlockSpec((1, D), lambda i: (0, 0)),
        ],
        out_specs=pl.BlockSpec((bm, D), lambda i: (i, 0)),
        out_shape=jax.ShapeDtypeStruct((m, D), jnp.float32),
    )(x, wt, b2)


def _sc_body(h_hbm, src_hbm, dst_hbm, agg_out, deg_out,
             agg_sh, deg_sh, src_v, dst_v, rows_v, ones_v, zer_v, zdeg_v, sem):
    c = lax.axis_index("c")
    s = lax.axis_index("s")
    w = c * NS + s
    row0 = s * ROWS_PER_TILE

    def fill_zer(i, carry):
        zer_v[i // 8, pl.ds((i % 8) * 16, 16)] = jnp.zeros((16,), jnp.float32)
        return carry
    lax.fori_loop(0, ZBLK * 8, fill_zer, 0)

    def fill_ones(i, carry):
        ones_v[pl.ds(i * 16, 16)] = jnp.ones((16,), jnp.float32)
        return carry
    lax.fori_loop(0, CHUNK // 16, fill_ones, 0)

    def fill_zdeg(i, carry):
        zdeg_v[pl.ds(i * 16, 16)] = jnp.zeros((16,), jnp.float32)
        return carry
    lax.fori_loop(0, ROWS_PER_TILE // 16, fill_zdeg, 0)

    def zero_slab(r, carry):
        pltpu.sync_copy(zer_v, agg_sh.at[pl.ds(row0 + r * ZBLK, ZBLK)])
        return carry
    lax.fori_loop(0, ROWS_PER_TILE // ZBLK, zero_slab, 0)
    pltpu.sync_copy(zdeg_v, deg_sh.at[pl.ds(row0, ROWS_PER_TILE)])

    plsc.subcore_barrier()

    pltpu.sync_copy(src_hbm.at[pl.ds(w * CHUNKS_PER_WORKER, CHUNKS_PER_WORKER)],
                    src_v)
    pltpu.sync_copy(dst_hbm.at[pl.ds(w * CHUNKS_PER_WORKER, CHUNKS_PER_WORKER)],
                    dst_v)

    def step(j, carry):
        pltpu.async_copy(h_hbm.at[src_v.at[j]], rows_v, sem).wait()
        pltpu.sync_copy(rows_v, agg_sh.at[dst_v.at[j]], add=True)
        pltpu.sync_copy(ones_v, deg_sh.at[dst_v.at[j]], add=True)
        return carry
    lax.fori_loop(0, CHUNKS_PER_WORKER, step, 0)

    plsc.subcore_barrier()

    pltpu.sync_copy(agg_sh.at[pl.ds(row0, ROWS_PER_TILE)],
                    agg_out.at[c, pl.ds(row0, ROWS_PER_TILE)])
    pltpu.sync_copy(deg_sh.at[pl.ds(row0, ROWS_PER_TILE)],
                    deg_out.at[c, pl.ds(row0, ROWS_PER_TILE)])


def _sc_aggregate(h, src2d, dst2d):
    mesh = plsc.VectorSubcoreMesh(core_axis_name="c", subcore_axis_name="s",
                                  num_cores=NC, num_subcores=NS)
    kern = pl.kernel(
        _sc_body,
        out_type=[
            jax.ShapeDtypeStruct((NC, AGG_ROWS, D), jnp.float32),
            jax.ShapeDtypeStruct((NC, AGG_ROWS), jnp.float32),
        ],
        mesh=mesh,
        scratch_types=[
            pltpu.VMEM_SHARED((AGG_ROWS, D), jnp.float32),
            pltpu.VMEM_SHARED((AGG_ROWS,), jnp.float32),
            pltpu.VMEM((CHUNKS_PER_WORKER, CHUNK), jnp.int32),
            pltpu.VMEM((CHUNKS_PER_WORKER, CHUNK), jnp.int32),
            pltpu.VMEM((CHUNK, D), jnp.float32),
            pltpu.VMEM((CHUNK,), jnp.float32),
            pltpu.VMEM((ZBLK, D), jnp.float32),
            pltpu.VMEM((ROWS_PER_TILE,), jnp.float32),
            pltpu.SemaphoreType.DMA,
        ],
    )
    return kern(h, src2d, dst2d)


def _combine_body(p_ref, d_ref, o_ref):
    d = jnp.maximum(d_ref[0] + d_ref[1], 1.0)
    o_ref[...] = (p_ref[0] + p_ref[1]) / d


def _combine(p, d3):
    bm = 1024
    return pl.pallas_call(
        _combine_body,
        grid=(AGG_ROWS // bm,),
        in_specs=[
            pl.BlockSpec((NC, bm, D), lambda i: (0, i, 0)),
            pl.BlockSpec((NC, bm, 1), lambda i: (0, i, 0)),
        ],
        out_specs=pl.BlockSpec((bm, D), lambda i: (i, 0)),
        out_shape=jax.ShapeDtypeStruct((AGG_ROWS, D), jnp.float32),
    )(p, d3)


def kernel(x, edge_index, W, b):
    n_edges = edge_index.shape[1]
    pad = E_PAD - n_edges
    dst = edge_index[0]
    src = edge_index[1]
    src_p = jnp.concatenate(
        [src, jnp.zeros((pad,), jnp.int32)]).reshape(E_PAD // CHUNK, CHUNK)
    dst_p = jnp.concatenate(
        [dst, jnp.full((pad,), DUMMY_ROW, jnp.int32)]).reshape(
            E_PAD // CHUNK, CHUNK)

    h = _linear(x, W.T, b.reshape(1, D))
    agg_p, deg_p = _sc_aggregate(h, src_p, dst_p)
    out_full = _combine(agg_p, deg_p.reshape(NC, AGG_ROWS, 1))
    return out_full[:NUM_NODES]


# speedup vs baseline: 4.1547x; 1.0865x over previous
import functools

import jax
import jax.numpy as jnp
from jax import lax
from jax.experimental import pallas as pl
from jax.experimental.pallas import tpu as pltpu
from jax.experimental.pallas import tpu_sc as plsc

D = 128
NUM_NODES = 10000

NC = 2
NS = 16
NW = NC * NS

CHUNK = 128
CHUNKS_PER_WORKER = 80
EDGES_PER_WORKER = CHUNK * CHUNKS_PER_WORKER
E_PAD = EDGES_PER_WORKER * NW
AGG_ROWS = 10240
ROWS_PER_TILE = AGG_ROWS // NS
DUMMY_ROW = NUM_NODES

SB = 16
NSB = CHUNKS_PER_WORKER // SB


def _linear_body(x_ref, wt_ref, b_ref, h_ref):
    h_ref[...] = (
        jnp.dot(x_ref[...], wt_ref[...], preferred_element_type=jnp.float32)
        + b_ref[...]
    )


def _linear(x, wt, b2):
    m = x.shape[0]
    bm = 1000
    return pl.pallas_call(
        _linear_body,
        grid=(m // bm,),
        in_specs=[
            pl.BlockSpec((bm, D), lambda i: (i, 0)),
            pl.BlockSpec((D, D), lambda i: (0, 0)),
            pl.BlockSpec((1, D), lambda i: (0, 0)),
        ],
        out_specs=pl.BlockSpec((bm, D), lambda i: (i, 0)),
        out_shape=jax.ShapeDtypeStruct((m, D), jnp.float32),
    )(x, wt, b2)


def _sc_body(h_hbm, src_hbm, dst_hbm, agg_out, deg_out,
             agg_sh, deg_sh, src_v, dst_v, rows0_v, rows1_v, ones_v,
             zdeg_v, sem0, sem1):
    c = lax.axis_index("c")
    s = lax.axis_index("s")
    w = c * NS + s
    row0 = s * ROWS_PER_TILE

    def fill_zrows(i, carry):
        rows0_v[i // 8, pl.ds((i % 8) * 16, 16)] = jnp.zeros((16,), jnp.float32)
        return carry
    lax.fori_loop(0, CHUNK * 8, fill_zrows, 0)

    def fill_ones(i, carry):
        ones_v[pl.ds(i * 16, 16)] = jnp.ones((16,), jnp.float32)
        return carry
    lax.fori_loop(0, CHUNK // 16, fill_ones, 0)

    def fill_zdeg(i, carry):
        zdeg_v[pl.ds(i * 16, 16)] = jnp.zeros((16,), jnp.float32)
        return carry
    lax.fori_loop(0, ROWS_PER_TILE // 16, fill_zdeg, 0)

    def zero_slab(r, carry):
        pltpu.sync_copy(rows0_v, agg_sh.at[pl.ds(row0 + r * CHUNK, CHUNK)])
        return carry
    lax.fori_loop(0, ROWS_PER_TILE // CHUNK, zero_slab, 0)
    pltpu.sync_copy(zdeg_v, deg_sh.at[pl.ds(row0, ROWS_PER_TILE)])

    plsc.subcore_barrier()

    base = w * CHUNKS_PER_WORKER

    def superblock(sb, carry):
        pltpu.sync_copy(src_hbm.at[pl.ds(base + sb * SB, SB)], src_v)
        pltpu.sync_copy(dst_hbm.at[pl.ds(base + sb * SB, SB)], dst_v)
        pltpu.async_copy(h_hbm.at[src_v.at[0]], rows0_v, sem0)

        def step2(i, c2):
            j0 = i * 2
            j1 = j0 + 1
            pltpu.async_copy(h_hbm.at[src_v.at[j1]], rows1_v, sem1)
            pltpu.make_async_copy(h_hbm.at[src_v.at[j0]], rows0_v, sem0).wait()
            pltpu.sync_copy(rows0_v, agg_sh.at[dst_v.at[j0]], add=True)
            pltpu.sync_copy(ones_v, deg_sh.at[dst_v.at[j0]], add=True)

            @pl.when(i < SB // 2 - 1)
            def _():
                pltpu.async_copy(h_hbm.at[src_v.at[j0 + 2]], rows0_v, sem0)

            pltpu.make_async_copy(h_hbm.at[src_v.at[j1]], rows1_v, sem1).wait()
            pltpu.sync_copy(rows1_v, agg_sh.at[dst_v.at[j1]], add=True)
            pltpu.sync_copy(ones_v, deg_sh.at[dst_v.at[j1]], add=True)
            return c2
        lax.fori_loop(0, SB // 2, step2, 0)
        return carry
    lax.fori_loop(0, NSB, superblock, 0)

    plsc.subcore_barrier()

    pltpu.sync_copy(agg_sh.at[pl.ds(row0, ROWS_PER_TILE)],
                    agg_out.at[c, pl.ds(row0, ROWS_PER_TILE)])
    pltpu.sync_copy(deg_sh.at[pl.ds(row0, ROWS_PER_TILE)],
                    deg_out.at[c, pl.ds(row0, ROWS_PER_TILE)])


def _sc_aggregate(h, src2d, dst2d):
    mesh = plsc.VectorSubcoreMesh(core_axis_name="c", subcore_axis_name="s",
                                  num_cores=NC, num_subcores=NS)
    kern = pl.kernel(
        _sc_body,
        out_type=[
            jax.ShapeDtypeStruct((NC, AGG_ROWS, D), jnp.float32),
            jax.ShapeDtypeStruct((NC, AGG_ROWS), jnp.float32),
        ],
        mesh=mesh,
        scratch_types=[
            pltpu.VMEM_SHARED((AGG_ROWS, D), jnp.float32),
            pltpu.VMEM_SHARED((AGG_ROWS,), jnp.float32),
            pltpu.VMEM((SB, CHUNK), jnp.int32),
            pltpu.VMEM((SB, CHUNK), jnp.int32),
            pltpu.VMEM((CHUNK, D), jnp.float32),
            pltpu.VMEM((CHUNK, D), jnp.float32),
            pltpu.VMEM((CHUNK,), jnp.float32),
            pltpu.VMEM((ROWS_PER_TILE,), jnp.float32),
            pltpu.SemaphoreType.DMA,
            pltpu.SemaphoreType.DMA,
        ],
    )
    return kern(h, src2d, dst2d)


def _combine_body(p_ref, d_ref, o_ref):
    d = jnp.maximum(d_ref[0] + d_ref[1], 1.0)
    o_ref[...] = (p_ref[0] + p_ref[1]) / d


def _combine(p, d3):
    bm = 1024
    return pl.pallas_call(
        _combine_body,
        grid=(AGG_ROWS // bm,),
        in_specs=[
            pl.BlockSpec((NC, bm, D), lambda i: (0, i, 0)),
            pl.BlockSpec((NC, bm, 1), lambda i: (0, i, 0)),
        ],
        out_specs=pl.BlockSpec((bm, D), lambda i: (i, 0)),
        out_shape=jax.ShapeDtypeStruct((AGG_ROWS, D), jnp.float32),
    )(p, d3)


def kernel(x, edge_index, W, b):
    n_edges = edge_index.shape[1]
    pad = E_PAD - n_edges
    dst = edge_index[0]
    src = edge_index[1]
    src_p = jnp.concatenate(
        [src, jnp.zeros((pad,), jnp.int32)]).reshape(E_PAD // CHUNK, CHUNK)
    dst_p = jnp.concatenate(
        [dst, jnp.full((pad,), DUMMY_ROW, jnp.int32)]).reshape(
            E_PAD // CHUNK, CHUNK)

    h = _linear(x, W.T, b.reshape(1, D))
    agg_p, deg_p = _sc_aggregate(h, src_p, dst_p)
    out_full = _combine(agg_p, deg_p.reshape(NC, AGG_ROWS, 1))
    return out_full[:NUM_NODES]


# speedup vs baseline: 4.2526x; 1.0235x over previous
import functools

import jax
import jax.numpy as jnp
from jax import lax
from jax.experimental import pallas as pl
from jax.experimental.pallas import tpu as pltpu
from jax.experimental.pallas import tpu_sc as plsc

D = 128
NUM_NODES = 10000

NC = 2
NS = 16
NW = NC * NS

CHUNK = 64
NBUF = 4
CHUNKS_PER_WORKER = 160
EDGES_PER_WORKER = CHUNK * CHUNKS_PER_WORKER
E_PAD = EDGES_PER_WORKER * NW
AGG_ROWS = 10240
ROWS_PER_TILE = AGG_ROWS // NS
DUMMY_ROW = NUM_NODES

SB = 32
NSB = CHUNKS_PER_WORKER // SB


def _linear_body(x_ref, wt_ref, b_ref, h_ref):
    h_ref[...] = (
        jnp.dot(x_ref[...], wt_ref[...], preferred_element_type=jnp.float32)
        + b_ref[...]
    )


def _linear(x, wt, b2):
    m = x.shape[0]
    bm = 1000
    return pl.pallas_call(
        _linear_body,
        grid=(m // bm,),
        in_specs=[
            pl.BlockSpec((bm, D), lambda i: (i, 0)),
            pl.BlockSpec((D, D), lambda i: (0, 0)),
            pl.BlockSpec((1, D), lambda i: (0, 0)),
        ],
        out_specs=pl.BlockSpec((bm, D), lambda i: (i, 0)),
        out_shape=jax.ShapeDtypeStruct((m, D), jnp.float32),
    )(x, wt, b2)


def _sc_body(h_hbm, src_hbm, dst_hbm, agg_out, deg_out,
             agg_sh, deg_sh, src_v, dst_v, rows0_v, rows1_v, rows2_v, rows3_v,
             ones_v, zdeg_v, sem0, sem1, sem2, sem3):
    rows = (rows0_v, rows1_v, rows2_v, rows3_v)
    sems = (sem0, sem1, sem2, sem3)
    c = lax.axis_index("c")
    s = lax.axis_index("s")
    w = c * NS + s
    row0 = s * ROWS_PER_TILE

    def fill_zrows(i, carry):
        rows0_v[i // 8, pl.ds((i % 8) * 16, 16)] = jnp.zeros((16,), jnp.float32)
        return carry
    lax.fori_loop(0, CHUNK * 8, fill_zrows, 0)

    def fill_ones(i, carry):
        ones_v[pl.ds(i * 16, 16)] = jnp.ones((16,), jnp.float32)
        return carry
    lax.fori_loop(0, CHUNK // 16, fill_ones, 0)

    def fill_zdeg(i, carry):
        zdeg_v[pl.ds(i * 16, 16)] = jnp.zeros((16,), jnp.float32)
        return carry
    lax.fori_loop(0, ROWS_PER_TILE // 16, fill_zdeg, 0)

    def zero_slab(r, carry):
        pltpu.sync_copy(rows0_v, agg_sh.at[pl.ds(row0 + r * CHUNK, CHUNK)])
        return carry
    lax.fori_loop(0, ROWS_PER_TILE // CHUNK, zero_slab, 0)
    pltpu.sync_copy(zdeg_v, deg_sh.at[pl.ds(row0, ROWS_PER_TILE)])

    plsc.subcore_barrier()

    base = w * CHUNKS_PER_WORKER

    def superblock(sb, carry):
        pltpu.sync_copy(src_hbm.at[pl.ds(base + sb * SB, SB)], src_v)
        pltpu.sync_copy(dst_hbm.at[pl.ds(base + sb * SB, SB)], dst_v)
        for k in range(NBUF - 1):
            pltpu.async_copy(h_hbm.at[src_v.at[k]], rows[k], sems[k])

        def quad(q, c2):
            for k in range(NBUF):
                j = q * NBUF + k
                pltpu.make_async_copy(h_hbm.at[src_v.at[j]], rows[k],
                                      sems[k]).wait()
                nxt = (k + NBUF - 1) % NBUF

                @pl.when(j + NBUF - 1 < SB)
                def _():
                    pltpu.async_copy(h_hbm.at[src_v.at[j + NBUF - 1]],
                                     rows[nxt], sems[nxt])

                pltpu.sync_copy(rows[k], agg_sh.at[dst_v.at[j]], add=True)
                pltpu.sync_copy(ones_v, deg_sh.at[dst_v.at[j]], add=True)
            return c2
        lax.fori_loop(0, SB // NBUF, quad, 0)
        return carry
    lax.fori_loop(0, NSB, superblock, 0)

    plsc.subcore_barrier()

    pltpu.sync_copy(agg_sh.at[pl.ds(row0, ROWS_PER_TILE)],
                    agg_out.at[c, pl.ds(row0, ROWS_PER_TILE)])
    pltpu.sync_copy(deg_sh.at[pl.ds(row0, ROWS_PER_TILE)],
                    deg_out.at[c, pl.ds(row0, ROWS_PER_TILE)])


def _sc_aggregate(h, src2d, dst2d):
    mesh = plsc.VectorSubcoreMesh(core_axis_name="c", subcore_axis_name="s",
                                  num_cores=NC, num_subcores=NS)
    kern = pl.kernel(
        _sc_body,
        out_type=[
            jax.ShapeDtypeStruct((NC, AGG_ROWS, D), jnp.float32),
            jax.ShapeDtypeStruct((NC, AGG_ROWS), jnp.float32),
        ],
        mesh=mesh,
        scratch_types=[
            pltpu.VMEM_SHARED((AGG_ROWS, D), jnp.float32),
            pltpu.VMEM_SHARED((AGG_ROWS,), jnp.float32),
            pltpu.VMEM((SB, CHUNK), jnp.int32),
            pltpu.VMEM((SB, CHUNK), jnp.int32),
            pltpu.VMEM((CHUNK, D), jnp.float32),
            pltpu.VMEM((CHUNK, D), jnp.float32),
            pltpu.VMEM((CHUNK, D), jnp.float32),
            pltpu.VMEM((CHUNK, D), jnp.float32),
            pltpu.VMEM((CHUNK,), jnp.float32),
            pltpu.VMEM((ROWS_PER_TILE,), jnp.float32),
            pltpu.SemaphoreType.DMA,
            pltpu.SemaphoreType.DMA,
            pltpu.SemaphoreType.DMA,
            pltpu.SemaphoreType.DMA,
        ],
    )
    return kern(h, src2d, dst2d)


def _combine_body(p_ref, d_ref, o_ref):
    d = jnp.maximum(d_ref[0] + d_ref[1], 1.0)
    o_ref[...] = (p_ref[0] + p_ref[1]) / d


def _combine(p, d3):
    bm = 1024
    return pl.pallas_call(
        _combine_body,
        grid=(AGG_ROWS // bm,),
        in_specs=[
            pl.BlockSpec((NC, bm, D), lambda i: (0, i, 0)),
            pl.BlockSpec((NC, bm, 1), lambda i: (0, i, 0)),
        ],
        out_specs=pl.BlockSpec((bm, D), lambda i: (i, 0)),
        out_shape=jax.ShapeDtypeStruct((AGG_ROWS, D), jnp.float32),
    )(p, d3)


def kernel(x, edge_index, W, b):
    n_edges = edge_index.shape[1]
    pad = E_PAD - n_edges
    dst = edge_index[0]
    src = edge_index[1]
    src_p = jnp.concatenate(
        [src, jnp.zeros((pad,), jnp.int32)]).reshape(E_PAD // CHUNK, CHUNK)
    dst_p = jnp.concatenate(
        [dst, jnp.full((pad,), DUMMY_ROW, jnp.int32)]).reshape(
            E_PAD // CHUNK, CHUNK)

    h = _linear(x, W.T, b.reshape(1, D))
    agg_p, deg_p = _sc_aggregate(h, src_p, dst_p)
    out_full = _combine(agg_p, deg_p.reshape(NC, AGG_ROWS, 1))
    return out_full[:NUM_NODES]
